# baseline (speedup 1.0000x reference)
import functools

import jax
import jax.numpy as jnp
from jax import lax
from jax.experimental import pallas as pl
from jax.experimental.pallas import tpu as pltpu
from jax.experimental.pallas import tpu_sc as plsc

F = 26
VOCAB = 100000
EMB = 16
BATCH = 16384
D1 = 32
D2 = 32

ROWS = BATCH * F
NC, NS = 2, 16
NW = NC * NS
RPW = ROWS // NW
CH = 3328
NCH = RPW // CH


def _gather_body(sec_hbm, first_hbm, idxf_hbm, idxb_hbm, sec_out, first_out,
                 idx_v, rows_v, f1_v, sem, fsem):
    wid = lax.axis_index("s") * NC + lax.axis_index("c")
    base = wid * RPW
    for c in range(NCH):
        off = base + c * CH
        pltpu.sync_copy(idxf_hbm.at[pl.ds(off, CH)], idx_v)
        pltpu.async_copy(sec_hbm.at[idx_v], rows_v, sem).wait()
        pltpu.sync_copy(rows_v, sec_out.at[pl.ds(off, CH)])
        pltpu.sync_copy(idxb_hbm.at[pl.ds(off, CH)], idx_v)
        pltpu.async_copy(first_hbm.at[idx_v], f1_v, fsem).wait()
        pltpu.sync_copy(f1_v, first_out.at[pl.ds(off, CH)])


@functools.lru_cache(maxsize=None)
def _make_gather():
    return pl.kernel(
        _gather_body,
        mesh=plsc.VectorSubcoreMesh(core_axis_name="c", subcore_axis_name="s"),
        compiler_params=pltpu.CompilerParams(use_tc_tiling_on_sc=False),
        out_type=(
            jax.ShapeDtypeStruct((ROWS, EMB), jnp.float32),
            jax.ShapeDtypeStruct((ROWS,), jnp.float32),
        ),
        scratch_types=[
            pltpu.VMEM((CH,), jnp.int32),
            pltpu.VMEM((CH, EMB), jnp.float32),
            pltpu.VMEM((CH,), jnp.float32),
            pltpu.SemaphoreType.DMA,
            pltpu.SemaphoreType.DMA,
        ],
    )


BS = 4096
NBLK = BATCH // BS


def _mlp_body(sec_ref, first_ref, xv_ref, w1a_ref, w1b3_ref, b1_ref,
              w2_ref, b2_ref, w3_ref, b3_ref, out_ref, acc):
    f = pl.program_id(1)

    @pl.when(f == 0)
    def _():
        acc[:, :] = jnp.zeros_like(acc)

    part = jnp.dot(sec_ref[:, :], w1b3_ref[0],
                   preferred_element_type=jnp.float32)
    lane = jax.lax.broadcasted_iota(jnp.int32, (BS, F), 1)
    xvf = jnp.sum(jnp.where(lane == f, xv_ref[:, :], 0.0), axis=1,
                  keepdims=True)
    acc[:, :] = acc[:, :] + part * xvf

    @pl.when(f == F - 1)
    def _():
        fo = first_ref[:, :] * xv_ref[:, :]
        acc1 = acc[:, :] + jnp.dot(fo, w1a_ref[:, :],
                                   preferred_element_type=jnp.float32)
        h = jnp.maximum(acc1 + b1_ref[:, :], 0.0)
        h = jnp.maximum(
            jnp.dot(h, w2_ref[:, :], preferred_element_type=jnp.float32)
            + b2_ref[:, :], 0.0)
        out_ref[:, :] = (
            jnp.dot(h, w3_ref[:, :], preferred_element_type=jnp.float32)
            + b3_ref[:, :])


def _mlp(sec_g, first_g, xv, w1a, w1b3, b1e, W2, b2, W3, b3):
    zero2 = lambda i, f: (0, 0)
    return pl.pallas_call(
        _mlp_body,
        grid=(NBLK, F),
        in_specs=[
            pl.BlockSpec((BS, EMB), lambda i, f: (f * NBLK + i, 0)),
            pl.BlockSpec((BS, F), lambda i, f: (i, 0)),
            pl.BlockSpec((BS, F), lambda i, f: (i, 0)),
            pl.BlockSpec((F, D1), lambda i, f: (0, 0)),
            pl.BlockSpec((1, EMB, D1), lambda i, f: (f, 0, 0)),
            pl.BlockSpec((1, D1), zero2),
            pl.BlockSpec((D1, D2), zero2),
            pl.BlockSpec((1, D2), zero2),
            pl.BlockSpec((D2, 1), zero2),
            pl.BlockSpec((1, 1), zero2),
        ],
        out_specs=pl.BlockSpec((BS, 1), lambda i, f: (i, 0)),
        out_shape=jax.ShapeDtypeStruct((BATCH, 1), jnp.float32),
        scratch_shapes=[pltpu.VMEM((BS, D1), jnp.float32)],
    )(sec_g, first_g, xv, w1a, w1b3, b1e, W2, b2, W3, b3)


def kernel(Xi, Xv, fm_bias, first_tables, second_tables, W1, b1, W2, b2, W3, b3):
    xi = Xi[:, :, 0].astype(jnp.int32)
    foff = jnp.arange(F, dtype=jnp.int32) * VOCAB
    idx_f = (xi.T + foff[:, None]).reshape(ROWS)
    idx_b = (xi + foff[None, :]).reshape(ROWS)
    sec_flat = second_tables.reshape(F * VOCAB, EMB)
    first_flat = first_tables.reshape(F * VOCAB)

    sec_g, first_g = _make_gather()(sec_flat, first_flat, idx_f, idx_b)

    w1a = W1[1:1 + F, :]
    w1b3 = W1[1 + F:, :].reshape(F, EMB, D1)
    b1e = (b1 + fm_bias * W1[0, :]).reshape(1, D1)
    out = _mlp(sec_g, first_g.reshape(BATCH, F), Xv.astype(jnp.float32),
               w1a, w1b3, b1e,
               W2, b2.reshape(1, D2), W3, b3.reshape(1, 1))
    return out.reshape(BATCH)
